# baseline (speedup 1.0000x reference)
import functools

import jax
import jax.numpy as jnp
from jax import lax
from jax.experimental import pallas as pl
from jax.experimental.pallas import tpu as pltpu
from jax.experimental.pallas import tpu_sc as plsc

N_NODES = 49152
N_EDGES = 1572864
IN_DIM = 25
HID = 6
OUT_DIM = 5
D_PAD = 8

NC = 2
NS = 16
EDGES_PER_CORE = N_EDGES // NC
EDGES_PER_SUB = EDGES_PER_CORE // NS
GROUP = 128
NGROUPS = EDGES_PER_SUB // GROUP
ROWS_PER_SUB = N_NODES // NS


SUP_E = 1024
K = SUP_E // GROUP
NSUP = EDGES_PER_SUB // SUP_E
NBUF = 4


def _spmm_sc(ij2, vals, x_pad, zero_rows):
    mesh = plsc.VectorSubcoreMesh(core_axis_name="c", subcore_axis_name="s")

    @functools.partial(
        pl.kernel,
        out_type=jax.ShapeDtypeStruct((NC, N_NODES, D_PAD), jnp.float32),
        mesh=mesh,
        scratch_types=[
            [pltpu.VMEM((2, K, GROUP), jnp.int32) for _ in range(NBUF)],
            [pltpu.VMEM((SUP_E,), jnp.float32) for _ in range(NBUF)],
            [pltpu.VMEM((SUP_E, D_PAD), jnp.float32) for _ in range(NBUF)],
            pltpu.VMEM_SHARED((N_NODES, D_PAD), jnp.float32),
            pltpu.SemaphoreType.DMA,
            pltpu.SemaphoreType.DMA,
            pltpu.SemaphoreType.DMA,
            pltpu.SemaphoreType.DMA,
        ],
        compiler_params=pltpu.CompilerParams(needs_layout_passes=False,
                                             use_tc_tiling_on_sc=False),
    )
    def spmm(ij_hbm, vals_hbm, x_hbm, z_hbm, out_hbm,
             ijb, valb, rows, acc, sem_i, sem_g, sem_s0, sem_s1):
        c = lax.axis_index("c")
        s = lax.axis_index("s")
        row0 = s * ROWS_PER_SUB
        base_e = c * EDGES_PER_CORE + s * EDGES_PER_SUB
        base_g = base_e // GROUP
        lanes = lax.iota(jnp.int32, 16)
        colv = [jnp.full((16,), cc, jnp.int32) for cc in range(D_PAD)]
        sem_s = [sem_s0, sem_s1]

        def idx_copies(g, u):
            grp = base_g + g * K
            return [
                pltpu.make_async_copy(
                    ij_hbm.at[:, pl.ds(grp, K), :], ijb[u], sem_i),
                pltpu.make_async_copy(
                    vals_hbm.at[pl.ds(base_e + g * SUP_E, SUP_E)],
                    valb[u], sem_i),
            ]

        def gather_copies(u):
            return [
                pltpu.make_async_copy(
                    x_hbm.at[ijb[u].at[1, k]],
                    rows[u].at[pl.ds(k * GROUP, GROUP)], sem_g)
                for k in range(K)
            ]

        def issue_scatter(u):
            for k in range(K):
                pltpu.async_copy(
                    rows[u].at[pl.ds(k * GROUP, GROUP)],
                    acc.at[ijb[u].at[0, k]], sem_s[u % 2], add=True)

        def wait_scatter(u):
            for k in range(K):
                pltpu.make_async_copy(
                    rows[u].at[pl.ds(k * GROUP, GROUP)],
                    acc.at[ijb[u].at[0, k]], sem_s[u % 2]).wait()

        def compute(u):
            @pl.loop(0, SUP_E // 16)
            def _(j):
                eidx = lanes + j * 16
                vv = valb[u][pl.ds(j * 16, 16)]
                for cc in range(D_PAD):
                    r = plsc.load_gather(rows[u], [eidx, colv[cc]])
                    plsc.store_scatter(rows[u], [eidx, colv[cc]], r * vv)

        for d in idx_copies(0, 0):
            d.start()
        pltpu.sync_copy(z_hbm, acc.at[pl.ds(row0, ROWS_PER_SUB)])
        plsc.subcore_barrier()
        for d in idx_copies(0, 0):
            d.wait()
        for d in gather_copies(0):
            d.start()
        for d in idx_copies(1, 1):
            d.start()

        @pl.loop(0, NSUP // NBUF)
        def _(t):
            for u in range(NBUF):
                g = t * NBUF + u

                @pl.when(g + 1 < NSUP)
                def _():
                    for d in idx_copies(g + 1, (u + 1) % NBUF):
                        d.wait()

                @pl.when(g >= 2)
                def _():
                    wait_scatter((u + 2) % NBUF)

                @pl.when(g + 1 < NSUP)
                def _():
                    for d in gather_copies((u + 1) % NBUF):
                        d.start()

                @pl.when(g + 2 < NSUP)
                def _():
                    for d in idx_copies(g + 2, (u + 2) % NBUF):
                        d.start()

                for d in gather_copies(u):
                    d.wait()
                compute(u)
                issue_scatter(u)

        wait_scatter((NSUP - 2) % NBUF)
        wait_scatter((NSUP - 1) % NBUF)

        plsc.subcore_barrier()
        pltpu.sync_copy(acc.at[pl.ds(row0, ROWS_PER_SUB)],
                        out_hbm.at[c, pl.ds(row0, ROWS_PER_SUB)])

    return spmm(ij2, vals, x_pad, zero_rows)


def _rne_bf16(x):
    u = plsc.bitcast(x, jnp.uint32)
    r = (u + jnp.uint32(0x7FFF) + ((u >> jnp.uint32(16)) & jnp.uint32(1)))
    r = r & jnp.uint32(0xFFFF0000)
    return plsc.bitcast(r, jnp.float32)


NW = NC * NS
RW = N_NODES // NW
_MESH = plsc.VectorSubcoreMesh(core_axis_name="c", subcore_axis_name="s")
_CP = pltpu.CompilerParams(needs_layout_passes=False,
                           use_tc_tiling_on_sc=False)


def _sc_layer1(f_flat, j_flat, ivec, w1p):
    @functools.partial(
        pl.kernel,
        out_type=jax.ShapeDtypeStruct((N_NODES, D_PAD), jnp.float32),
        mesh=_MESH,
        scratch_types=[
            pltpu.VMEM((RW * IN_DIM,), jnp.float32),
            pltpu.VMEM((RW * IN_DIM,), jnp.float32),
            pltpu.VMEM((RW, D_PAD), jnp.float32),
            pltpu.VMEM((IN_DIM * 16,), jnp.float32),
            pltpu.VMEM((16,), jnp.float32),
        ],
        compiler_params=_CP,
    )
    def l1(f_hbm, j_hbm, iv_hbm, w_hbm, out_hbm, fb, jb, ob, wb, ivb):
        w = lax.axis_index("s") * NC + lax.axis_index("c")
        r0 = w * RW
        pltpu.sync_copy(f_hbm.at[pl.ds(r0 * IN_DIM, RW * IN_DIM)], fb)
        pltpu.sync_copy(j_hbm.at[pl.ds(r0 * IN_DIM, RW * IN_DIM)], jb)
        pltpu.sync_copy(w_hbm, wb)
        pltpu.sync_copy(iv_hbm, ivb)
        iv = ivb[...]

        @pl.loop(0, RW * IN_DIM // 16)
        def _(u):
            sl = pl.ds(u * 16, 16)
            jb[sl] = _rne_bf16(fb[sl] + jb[sl] * iv)

        lanes = lax.iota(jnp.int32, 16)
        half = lanes >> 3
        colp = lanes & 7
        h25 = half * IN_DIM
        w1r = [wb[pl.ds(16 * k, 16)] for k in range(IN_DIM)]

        @pl.loop(0, RW // 2)
        def _(r):
            idx = 2 * r * IN_DIM + h25
            acc = w1r[0] * plsc.load_gather(jb, [idx])
            for k in range(1, IN_DIM):
                idx = idx + 1
                acc = acc + w1r[k] * plsc.load_gather(jb, [idx])
            plsc.store_scatter(ob, [2 * r + half, colp], acc)

        pltpu.sync_copy(ob, out_hbm.at[pl.ds(r0, RW)])

    return l1(f_flat, j_flat, ivec, w1p)


def _sc_mid(p_flat, b1p, w2p):
    @functools.partial(
        pl.kernel,
        out_type=jax.ShapeDtypeStruct((N_NODES, D_PAD), jnp.float32),
        mesh=_MESH,
        scratch_types=[
            pltpu.VMEM((2 * RW * D_PAD,), jnp.float32),
            pltpu.VMEM((RW, D_PAD), jnp.float32),
            pltpu.VMEM((D_PAD * 16,), jnp.float32),
            pltpu.VMEM((16,), jnp.float32),
        ],
        compiler_params=_CP,
    )
    def mid(p_hbm, b_hbm, w_hbm, out_hbm, pb, ob, wb, bb):
        w = lax.axis_index("s") * NC + lax.axis_index("c")
        r0 = w * RW
        nwords = RW * D_PAD
        pltpu.sync_copy(p_hbm.at[pl.ds(r0 * D_PAD, nwords)], pb.at[pl.ds(0, nwords)])
        pltpu.sync_copy(p_hbm.at[pl.ds(N_NODES * D_PAD + r0 * D_PAD, nwords)],
                        pb.at[pl.ds(nwords, nwords)])
        pltpu.sync_copy(w_hbm, wb)
        pltpu.sync_copy(b_hbm, bb)
        bv = bb[...]

        @pl.loop(0, nwords // 16)
        def _(u):
            sl = pl.ds(u * 16, 16)
            pb[sl] = _rne_bf16(
                jnp.maximum(pb[sl] + pb[pl.ds(nwords + u * 16, 16)] + bv, 0.0))

        lanes = lax.iota(jnp.int32, 16)
        half = lanes >> 3
        colp = lanes & 7
        h8 = half * D_PAD
        w2r = [wb[pl.ds(16 * k, 16)] for k in range(D_PAD)]

        @pl.loop(0, RW // 2)
        def _(r):
            idx = 2 * r * D_PAD + h8
            acc = w2r[0] * plsc.load_gather(pb, [idx])
            for k in range(1, D_PAD):
                idx = idx + 1
                acc = acc + w2r[k] * plsc.load_gather(pb, [idx])
            plsc.store_scatter(ob, [2 * r + half, colp], acc)

        pltpu.sync_copy(ob, out_hbm.at[pl.ds(r0, RW)])

    return mid(p_flat, b1p, w2p)


def _sc_final(p_flat, b2p):
    @functools.partial(
        pl.kernel,
        out_type=jax.ShapeDtypeStruct((N_NODES * D_PAD,), jnp.float32),
        mesh=_MESH,
        scratch_types=[
            pltpu.VMEM((2 * RW * D_PAD,), jnp.float32),
            pltpu.VMEM((16,), jnp.float32),
        ],
        compiler_params=_CP,
    )
    def fin(p_hbm, b_hbm, out_hbm, pb, bb):
        w = lax.axis_index("s") * NC + lax.axis_index("c")
        r0 = w * RW
        nwords = RW * D_PAD
        pltpu.sync_copy(p_hbm.at[pl.ds(r0 * D_PAD, nwords)], pb.at[pl.ds(0, nwords)])
        pltpu.sync_copy(p_hbm.at[pl.ds(N_NODES * D_PAD + r0 * D_PAD, nwords)],
                        pb.at[pl.ds(nwords, nwords)])
        pltpu.sync_copy(b_hbm, bb)
        bv = bb[...]

        @pl.loop(0, nwords // 16)
        def _(u):
            sl = pl.ds(u * 16, 16)
            x = pb[sl] + pb[pl.ds(nwords + u * 16, 16)] + bv
            pb[sl] = 1.0 / (1.0 + jnp.exp(-x))

        pltpu.sync_copy(pb.at[pl.ds(0, nwords)],
                        out_hbm.at[pl.ds(r0 * D_PAD, nwords)])

    return fin(p_flat, b2p)


def kernel(adjacency_indices, adjacency_values, conv1, conv2, i, W1, b1, W2, b2):
    kh, kw = conv1.shape[2], conv1.shape[3]
    x1 = conv1.reshape(-1, kh * kw)
    x2 = conv2.reshape(-1, kh * kw)
    feature = jnp.concatenate([x1, x2], axis=0)
    def _noise():
        return jax.random.uniform(jax.random.key(42), feature.shape,
                                  dtype=jnp.float32) * 2.0 - 1.0
    try:
        with jax.ensure_compile_time_eval():
            junyun = _noise()
    except Exception:
        junyun = _noise()
    ij2 = adjacency_indices.reshape(2, N_EDGES // GROUP, GROUP)

    w1_pad = jnp.pad(W1, ((0, 0), (0, D_PAD - HID)))
    w1_pad = w1_pad.astype(jnp.bfloat16).astype(jnp.float32)
    w1p = jnp.concatenate([w1_pad, w1_pad], axis=1).reshape(-1)
    b1p = jnp.tile(jnp.pad(b1, (0, D_PAD - HID)), 2)
    w2_pad = jnp.pad(W2, ((0, D_PAD - HID), (0, D_PAD - OUT_DIM)))
    w2_pad = w2_pad.astype(jnp.bfloat16).astype(jnp.float32)
    w2p = jnp.concatenate([w2_pad, w2_pad], axis=1).reshape(-1)
    b2p = jnp.tile(jnp.pad(b2, (0, D_PAD - OUT_DIM)), 2)
    ivec = jnp.broadcast_to(i, (16,))
    zero_rows = jnp.zeros((ROWS_PER_SUB, D_PAD), jnp.float32)

    support = _sc_layer1(feature.reshape(-1), junyun.reshape(-1), ivec, w1p)
    partials = _spmm_sc(ij2, adjacency_values, support, zero_rows)
    support2 = _sc_mid(partials.reshape(-1), b1p, w2p)
    partials2 = _spmm_sc(ij2, adjacency_values, support2, zero_rows)
    out = _sc_final(partials2.reshape(-1), b2p)
    return out.reshape(N_NODES, D_PAD)[:, :OUT_DIM]


# speedup vs baseline: 1.3893x; 1.3893x over previous
import functools

import jax
import jax.numpy as jnp
from jax import lax
from jax.experimental import pallas as pl
from jax.experimental.pallas import tpu as pltpu
from jax.experimental.pallas import tpu_sc as plsc

N_NODES = 49152
N_EDGES = 1572864
IN_DIM = 25
HID = 6
OUT_DIM = 5
D_PAD = 8

NC = 2
NS = 16
EDGES_PER_CORE = N_EDGES // NC
EDGES_PER_SUB = EDGES_PER_CORE // NS
GROUP = 128
NGROUPS = EDGES_PER_SUB // GROUP
ROWS_PER_SUB = N_NODES // NS


SUP_E = 1024
K = SUP_E // GROUP
NSUP = EDGES_PER_SUB // SUP_E
NBUF = 4


def _spmm_sc(ij2, vals, x_pad, zero_rows, ncols):
    mesh = plsc.VectorSubcoreMesh(core_axis_name="c", subcore_axis_name="s")

    @functools.partial(
        pl.kernel,
        out_type=jax.ShapeDtypeStruct((NC, N_NODES, D_PAD), jnp.float32),
        mesh=mesh,
        scratch_types=[
            [pltpu.VMEM((2, K, GROUP), jnp.int32) for _ in range(NBUF)],
            [pltpu.VMEM((SUP_E,), jnp.float32) for _ in range(NBUF)],
            [pltpu.VMEM((SUP_E, D_PAD), jnp.float32) for _ in range(NBUF)],
            pltpu.VMEM_SHARED((N_NODES, D_PAD), jnp.float32),
            pltpu.SemaphoreType.DMA,
            pltpu.SemaphoreType.DMA,
            pltpu.SemaphoreType.DMA,
            pltpu.SemaphoreType.DMA,
        ],
        compiler_params=pltpu.CompilerParams(needs_layout_passes=False,
                                             use_tc_tiling_on_sc=False),
    )
    def spmm(ij_hbm, vals_hbm, x_hbm, z_hbm, out_hbm,
             ijb, valb, rows, acc, sem_i, sem_g, sem_s0, sem_s1):
        c = lax.axis_index("c")
        s = lax.axis_index("s")
        row0 = s * ROWS_PER_SUB
        base_e = c * EDGES_PER_CORE + s * EDGES_PER_SUB
        base_g = base_e // GROUP
        lanes = lax.iota(jnp.int32, 16)
        colv = [jnp.full((16,), cc, jnp.int32) for cc in range(ncols)]
        sem_s = [sem_s0, sem_s1]

        def idx_copies(g, u):
            grp = base_g + g * K
            return [
                pltpu.make_async_copy(
                    ij_hbm.at[:, pl.ds(grp, K), :], ijb[u], sem_i),
                pltpu.make_async_copy(
                    vals_hbm.at[pl.ds(base_e + g * SUP_E, SUP_E)],
                    valb[u], sem_i),
            ]

        def gather_copies(u):
            return [
                pltpu.make_async_copy(
                    x_hbm.at[ijb[u].at[1, k]],
                    rows[u].at[pl.ds(k * GROUP, GROUP)], sem_g)
                for k in range(K)
            ]

        def issue_scatter(u):
            for k in range(K):
                pltpu.async_copy(
                    rows[u].at[pl.ds(k * GROUP, GROUP)],
                    acc.at[ijb[u].at[0, k]], sem_s[u % 2], add=True)

        def wait_scatter(u):
            for k in range(K):
                pltpu.make_async_copy(
                    rows[u].at[pl.ds(k * GROUP, GROUP)],
                    acc.at[ijb[u].at[0, k]], sem_s[u % 2]).wait()

        def compute(u):
            @plsc.parallel_loop(0, SUP_E // 16, unroll=4)
            def _(j):
                eidx = lanes + j * 16
                vv = valb[u][pl.ds(j * 16, 16)]
                for cc in range(ncols):
                    r = plsc.load_gather(rows[u], [eidx, colv[cc]])
                    plsc.store_scatter(rows[u], [eidx, colv[cc]], r * vv)

        for d in idx_copies(0, 0):
            d.start()
        pltpu.sync_copy(z_hbm, acc.at[pl.ds(row0, ROWS_PER_SUB)])
        plsc.subcore_barrier()
        for d in idx_copies(0, 0):
            d.wait()
        for d in gather_copies(0):
            d.start()
        for d in idx_copies(1, 1):
            d.start()

        @pl.loop(0, NSUP // NBUF)
        def _(t):
            for u in range(NBUF):
                g = t * NBUF + u

                @pl.when(g + 1 < NSUP)
                def _():
                    for d in idx_copies(g + 1, (u + 1) % NBUF):
                        d.wait()

                @pl.when(g >= 2)
                def _():
                    wait_scatter((u + 2) % NBUF)

                @pl.when(g + 1 < NSUP)
                def _():
                    for d in gather_copies((u + 1) % NBUF):
                        d.start()

                @pl.when(g + 2 < NSUP)
                def _():
                    for d in idx_copies(g + 2, (u + 2) % NBUF):
                        d.start()

                for d in gather_copies(u):
                    d.wait()
                compute(u)
                issue_scatter(u)

        wait_scatter((NSUP - 2) % NBUF)
        wait_scatter((NSUP - 1) % NBUF)

        plsc.subcore_barrier()
        pltpu.sync_copy(acc.at[pl.ds(row0, ROWS_PER_SUB)],
                        out_hbm.at[c, pl.ds(row0, ROWS_PER_SUB)])

    return spmm(ij2, vals, x_pad, zero_rows)


def _rne_bf16(x):
    u = plsc.bitcast(x, jnp.uint32)
    r = (u + jnp.uint32(0x7FFF) + ((u >> jnp.uint32(16)) & jnp.uint32(1)))
    r = r & jnp.uint32(0xFFFF0000)
    return plsc.bitcast(r, jnp.float32)


NW = NC * NS
RW = N_NODES // NW
_MESH = plsc.VectorSubcoreMesh(core_axis_name="c", subcore_axis_name="s")
_CP = pltpu.CompilerParams(needs_layout_passes=False,
                           use_tc_tiling_on_sc=False)


def _sc_layer1(f_flat, j_flat, ivec, w1p):
    @functools.partial(
        pl.kernel,
        out_type=jax.ShapeDtypeStruct((N_NODES, D_PAD), jnp.float32),
        mesh=_MESH,
        scratch_types=[
            pltpu.VMEM((RW * IN_DIM,), jnp.float32),
            pltpu.VMEM((RW * IN_DIM,), jnp.float32),
            pltpu.VMEM((RW, D_PAD), jnp.float32),
            pltpu.VMEM((IN_DIM * 16,), jnp.float32),
            pltpu.VMEM((16,), jnp.float32),
        ],
        compiler_params=_CP,
    )
    def l1(f_hbm, j_hbm, iv_hbm, w_hbm, out_hbm, fb, jb, ob, wb, ivb):
        w = lax.axis_index("s") * NC + lax.axis_index("c")
        r0 = w * RW
        pltpu.sync_copy(f_hbm.at[pl.ds(r0 * IN_DIM, RW * IN_DIM)], fb)
        pltpu.sync_copy(j_hbm.at[pl.ds(r0 * IN_DIM, RW * IN_DIM)], jb)
        pltpu.sync_copy(w_hbm, wb)
        pltpu.sync_copy(iv_hbm, ivb)
        iv = ivb[...]

        @pl.loop(0, RW * IN_DIM // 16)
        def _(u):
            sl = pl.ds(u * 16, 16)
            jb[sl] = _rne_bf16(fb[sl] + jb[sl] * iv)

        lanes = lax.iota(jnp.int32, 16)
        half = lanes >> 3
        colp = lanes & 7
        h25 = half * IN_DIM
        w1r = [wb[pl.ds(16 * k, 16)] for k in range(IN_DIM)]

        @pl.loop(0, RW // 2)
        def _(r):
            idx = 2 * r * IN_DIM + h25
            acc = w1r[0] * plsc.load_gather(jb, [idx])
            for k in range(1, IN_DIM):
                idx = idx + 1
                acc = acc + w1r[k] * plsc.load_gather(jb, [idx])
            plsc.store_scatter(ob, [2 * r + half, colp], acc)

        pltpu.sync_copy(ob, out_hbm.at[pl.ds(r0, RW)])

    return l1(f_flat, j_flat, ivec, w1p)


def _sc_mid(p_flat, b1p, w2p):
    @functools.partial(
        pl.kernel,
        out_type=jax.ShapeDtypeStruct((N_NODES, D_PAD), jnp.float32),
        mesh=_MESH,
        scratch_types=[
            pltpu.VMEM((2 * RW * D_PAD,), jnp.float32),
            pltpu.VMEM((RW, D_PAD), jnp.float32),
            pltpu.VMEM((D_PAD * 16,), jnp.float32),
            pltpu.VMEM((16,), jnp.float32),
        ],
        compiler_params=_CP,
    )
    def mid(p_hbm, b_hbm, w_hbm, out_hbm, pb, ob, wb, bb):
        w = lax.axis_index("s") * NC + lax.axis_index("c")
        r0 = w * RW
        nwords = RW * D_PAD
        pltpu.sync_copy(p_hbm.at[pl.ds(r0 * D_PAD, nwords)], pb.at[pl.ds(0, nwords)])
        pltpu.sync_copy(p_hbm.at[pl.ds(N_NODES * D_PAD + r0 * D_PAD, nwords)],
                        pb.at[pl.ds(nwords, nwords)])
        pltpu.sync_copy(w_hbm, wb)
        pltpu.sync_copy(b_hbm, bb)
        bv = bb[...]

        @pl.loop(0, nwords // 16)
        def _(u):
            sl = pl.ds(u * 16, 16)
            pb[sl] = _rne_bf16(
                jnp.maximum(pb[sl] + pb[pl.ds(nwords + u * 16, 16)] + bv, 0.0))

        lanes = lax.iota(jnp.int32, 16)
        half = lanes >> 3
        colp = lanes & 7
        h8 = half * D_PAD
        w2r = [wb[pl.ds(16 * k, 16)] for k in range(D_PAD)]

        @pl.loop(0, RW // 2)
        def _(r):
            idx = 2 * r * D_PAD + h8
            acc = w2r[0] * plsc.load_gather(pb, [idx])
            for k in range(1, D_PAD):
                idx = idx + 1
                acc = acc + w2r[k] * plsc.load_gather(pb, [idx])
            plsc.store_scatter(ob, [2 * r + half, colp], acc)

        pltpu.sync_copy(ob, out_hbm.at[pl.ds(r0, RW)])

    return mid(p_flat, b1p, w2p)


def _sc_final(p_flat, b2p):
    @functools.partial(
        pl.kernel,
        out_type=jax.ShapeDtypeStruct((N_NODES * D_PAD,), jnp.float32),
        mesh=_MESH,
        scratch_types=[
            pltpu.VMEM((2 * RW * D_PAD,), jnp.float32),
            pltpu.VMEM((16,), jnp.float32),
        ],
        compiler_params=_CP,
    )
    def fin(p_hbm, b_hbm, out_hbm, pb, bb):
        w = lax.axis_index("s") * NC + lax.axis_index("c")
        r0 = w * RW
        nwords = RW * D_PAD
        pltpu.sync_copy(p_hbm.at[pl.ds(r0 * D_PAD, nwords)], pb.at[pl.ds(0, nwords)])
        pltpu.sync_copy(p_hbm.at[pl.ds(N_NODES * D_PAD + r0 * D_PAD, nwords)],
                        pb.at[pl.ds(nwords, nwords)])
        pltpu.sync_copy(b_hbm, bb)
        bv = bb[...]

        @pl.loop(0, nwords // 16)
        def _(u):
            sl = pl.ds(u * 16, 16)
            x = pb[sl] + pb[pl.ds(nwords + u * 16, 16)] + bv
            pb[sl] = 1.0 / (1.0 + jnp.exp(-x))

        pltpu.sync_copy(pb.at[pl.ds(0, nwords)],
                        out_hbm.at[pl.ds(r0 * D_PAD, nwords)])

    return fin(p_flat, b2p)


def kernel(adjacency_indices, adjacency_values, conv1, conv2, i, W1, b1, W2, b2):
    kh, kw = conv1.shape[2], conv1.shape[3]
    x1 = conv1.reshape(-1, kh * kw)
    x2 = conv2.reshape(-1, kh * kw)
    feature = jnp.concatenate([x1, x2], axis=0)
    def _noise():
        return jax.random.uniform(jax.random.key(42), feature.shape,
                                  dtype=jnp.float32) * 2.0 - 1.0
    try:
        with jax.ensure_compile_time_eval():
            junyun = _noise()
    except Exception:
        junyun = _noise()
    ij2 = adjacency_indices.reshape(2, N_EDGES // GROUP, GROUP)

    w1_pad = jnp.pad(W1, ((0, 0), (0, D_PAD - HID)))
    w1_pad = w1_pad.astype(jnp.bfloat16).astype(jnp.float32)
    w1p = jnp.concatenate([w1_pad, w1_pad], axis=1).reshape(-1)
    b1p = jnp.tile(jnp.pad(b1, (0, D_PAD - HID)), 2)
    w2_pad = jnp.pad(W2, ((0, D_PAD - HID), (0, D_PAD - OUT_DIM)))
    w2_pad = w2_pad.astype(jnp.bfloat16).astype(jnp.float32)
    w2p = jnp.concatenate([w2_pad, w2_pad], axis=1).reshape(-1)
    b2p = jnp.tile(jnp.pad(b2, (0, D_PAD - OUT_DIM)), 2)
    ivec = jnp.broadcast_to(i, (16,))
    zero_rows = jnp.zeros((ROWS_PER_SUB, D_PAD), jnp.float32)

    support = _sc_layer1(feature.reshape(-1), junyun.reshape(-1), ivec, w1p)
    partials = _spmm_sc(ij2, adjacency_values, support, zero_rows, HID)
    support2 = _sc_mid(partials.reshape(-1), b1p, w2p)
    partials2 = _spmm_sc(ij2, adjacency_values, support2, zero_rows, OUT_DIM)
    out = _sc_final(partials2.reshape(-1), b2p)
    return out.reshape(N_NODES, D_PAD)[:, :OUT_DIM]


# speedup vs baseline: 1.5779x; 1.1358x over previous
import functools

import jax
import jax.numpy as jnp
from jax import lax
from jax.experimental import pallas as pl
from jax.experimental.pallas import tpu as pltpu
from jax.experimental.pallas import tpu_sc as plsc

N_NODES = 49152
N_EDGES = 1572864
IN_DIM = 25
HID = 6
OUT_DIM = 5
D_PAD = 8

NC = 2
NS = 16
EDGES_PER_CORE = N_EDGES // NC
EDGES_PER_SUB = EDGES_PER_CORE // NS
GROUP = 128
NGROUPS = EDGES_PER_SUB // GROUP
ROWS_PER_SUB = N_NODES // NS


SUP_E = 1024
K = SUP_E // GROUP
NSUP = EDGES_PER_SUB // SUP_E
NBUF = 4


def _spmm_sc(ij2, vals, x_pad, zero_rows, ncols):
    mesh = plsc.VectorSubcoreMesh(core_axis_name="c", subcore_axis_name="s")

    @functools.partial(
        pl.kernel,
        out_type=jax.ShapeDtypeStruct((NC, N_NODES, D_PAD), jnp.float32),
        mesh=mesh,
        scratch_types=[
            [pltpu.VMEM((2, K, GROUP), jnp.int32) for _ in range(NBUF)],
            [pltpu.VMEM((SUP_E,), jnp.float32) for _ in range(NBUF)],
            [pltpu.VMEM((SUP_E, D_PAD), jnp.float32) for _ in range(NBUF)],
            pltpu.VMEM_SHARED((N_NODES, D_PAD), jnp.float32),
            pltpu.SemaphoreType.DMA,
            pltpu.SemaphoreType.DMA,
            pltpu.SemaphoreType.DMA,
            pltpu.SemaphoreType.DMA,
        ],
        compiler_params=pltpu.CompilerParams(needs_layout_passes=False,
                                             use_tc_tiling_on_sc=False),
    )
    def spmm(ij_hbm, vals_hbm, x_hbm, z_hbm, out_hbm,
             ijb, valb, rows, acc, sem_i, sem_g, sem_s0, sem_s1):
        c = lax.axis_index("c")
        s = lax.axis_index("s")
        row0 = s * ROWS_PER_SUB
        base_e = c * EDGES_PER_CORE + s * EDGES_PER_SUB
        base_g = base_e // GROUP
        lanes = lax.iota(jnp.int32, 16)
        colv = [jnp.full((16,), cc, jnp.int32) for cc in range(ncols)]
        sem_s = [sem_s0, sem_s1]

        def idx_copies(g, u):
            grp = base_g + g * K
            return [
                pltpu.make_async_copy(
                    ij_hbm.at[:, pl.ds(grp, K), :], ijb[u], sem_i),
                pltpu.make_async_copy(
                    vals_hbm.at[pl.ds(base_e + g * SUP_E, SUP_E)],
                    valb[u], sem_i),
            ]

        def gather_copies(u):
            return [
                pltpu.make_async_copy(
                    x_hbm.at[ijb[u].at[1, k]],
                    rows[u].at[pl.ds(k * GROUP, GROUP)], sem_g)
                for k in range(K)
            ]

        def issue_scatter(u):
            for k in range(K):
                pltpu.async_copy(
                    rows[u].at[pl.ds(k * GROUP, GROUP)],
                    acc.at[ijb[u].at[0, k]], sem_s[u % 2], add=True)

        def wait_scatter(u):
            for k in range(K):
                pltpu.make_async_copy(
                    rows[u].at[pl.ds(k * GROUP, GROUP)],
                    acc.at[ijb[u].at[0, k]], sem_s[u % 2]).wait()

        def compute(u):
            @plsc.parallel_loop(0, SUP_E // 16, unroll=4)
            def _(j):
                eidx = lanes + j * 16
                vv = valb[u][pl.ds(j * 16, 16)]
                for cc in range(ncols):
                    r = plsc.load_gather(rows[u], [eidx, colv[cc]])
                    plsc.store_scatter(rows[u], [eidx, colv[cc]], r * vv)

        for d in idx_copies(0, 0):
            d.start()
        pltpu.sync_copy(z_hbm, acc.at[pl.ds(row0, ROWS_PER_SUB)])
        plsc.subcore_barrier()
        for d in idx_copies(0, 0):
            d.wait()
        for d in gather_copies(0):
            d.start()
        for d in idx_copies(1, 1):
            d.start()

        @pl.loop(0, NSUP // NBUF)
        def _(t):
            for u in range(NBUF):
                g = t * NBUF + u

                @pl.when(g + 1 < NSUP)
                def _():
                    for d in idx_copies(g + 1, (u + 1) % NBUF):
                        d.wait()

                @pl.when(g >= 2)
                def _():
                    wait_scatter((u + 2) % NBUF)

                @pl.when(g + 1 < NSUP)
                def _():
                    for d in gather_copies((u + 1) % NBUF):
                        d.start()

                @pl.when(g + 2 < NSUP)
                def _():
                    for d in idx_copies(g + 2, (u + 2) % NBUF):
                        d.start()

                for d in gather_copies(u):
                    d.wait()
                compute(u)
                issue_scatter(u)

        wait_scatter((NSUP - 2) % NBUF)
        wait_scatter((NSUP - 1) % NBUF)

        plsc.subcore_barrier()
        pltpu.sync_copy(acc.at[pl.ds(row0, ROWS_PER_SUB)],
                        out_hbm.at[c, pl.ds(row0, ROWS_PER_SUB)])

    return spmm(ij2, vals, x_pad, zero_rows)


def _rne_bf16(x):
    u = plsc.bitcast(x, jnp.uint32)
    r = (u + jnp.uint32(0x7FFF) + ((u >> jnp.uint32(16)) & jnp.uint32(1)))
    r = r & jnp.uint32(0xFFFF0000)
    return plsc.bitcast(r, jnp.float32)


NW = NC * NS
RW = N_NODES // NW
_MESH = plsc.VectorSubcoreMesh(core_axis_name="c", subcore_axis_name="s")
_CP = pltpu.CompilerParams(needs_layout_passes=False,
                           use_tc_tiling_on_sc=False)


def _sc_layer1(f_flat, j_flat, ivec, w1p):
    @functools.partial(
        pl.kernel,
        out_type=jax.ShapeDtypeStruct((N_NODES, D_PAD), jnp.float32),
        mesh=_MESH,
        scratch_types=[
            pltpu.VMEM((RW * IN_DIM,), jnp.float32),
            pltpu.VMEM((RW * IN_DIM,), jnp.float32),
            pltpu.VMEM((RW, D_PAD), jnp.float32),
            pltpu.VMEM((IN_DIM * 16,), jnp.float32),
            pltpu.VMEM((16,), jnp.float32),
        ],
        compiler_params=_CP,
    )
    def l1(f_hbm, j_hbm, iv_hbm, w_hbm, out_hbm, fb, jb, ob, wb, ivb):
        w = lax.axis_index("s") * NC + lax.axis_index("c")
        r0 = w * RW
        pltpu.sync_copy(f_hbm.at[pl.ds(r0 * IN_DIM, RW * IN_DIM)], fb)
        pltpu.sync_copy(j_hbm.at[pl.ds(r0 * IN_DIM, RW * IN_DIM)], jb)
        pltpu.sync_copy(w_hbm, wb)
        pltpu.sync_copy(iv_hbm, ivb)
        iv = ivb[...]

        @plsc.parallel_loop(0, RW * IN_DIM // 16, unroll=4)
        def _(u):
            sl = pl.ds(u * 16, 16)
            jb[sl] = _rne_bf16(fb[sl] + jb[sl] * iv)

        lanes = lax.iota(jnp.int32, 16)
        half = lanes >> 3
        colp = lanes & 7
        h25 = half * IN_DIM
        w1r = [wb[pl.ds(16 * k, 16)] for k in range(IN_DIM)]

        @plsc.parallel_loop(0, RW // 2, unroll=4)
        def _(r):
            idx = 2 * r * IN_DIM + h25
            acc = w1r[0] * plsc.load_gather(jb, [idx])
            for k in range(1, IN_DIM):
                idx = idx + 1
                acc = acc + w1r[k] * plsc.load_gather(jb, [idx])
            plsc.store_scatter(ob, [2 * r + half, colp], acc)

        pltpu.sync_copy(ob, out_hbm.at[pl.ds(r0, RW)])

    return l1(f_flat, j_flat, ivec, w1p)


def _sc_mid(p_flat, b1p, w2p):
    @functools.partial(
        pl.kernel,
        out_type=jax.ShapeDtypeStruct((N_NODES, D_PAD), jnp.float32),
        mesh=_MESH,
        scratch_types=[
            pltpu.VMEM((2 * RW * D_PAD,), jnp.float32),
            pltpu.VMEM((RW, D_PAD), jnp.float32),
            pltpu.VMEM((D_PAD * 16,), jnp.float32),
            pltpu.VMEM((16,), jnp.float32),
        ],
        compiler_params=_CP,
    )
    def mid(p_hbm, b_hbm, w_hbm, out_hbm, pb, ob, wb, bb):
        w = lax.axis_index("s") * NC + lax.axis_index("c")
        r0 = w * RW
        nwords = RW * D_PAD
        pltpu.sync_copy(p_hbm.at[pl.ds(r0 * D_PAD, nwords)], pb.at[pl.ds(0, nwords)])
        pltpu.sync_copy(p_hbm.at[pl.ds(N_NODES * D_PAD + r0 * D_PAD, nwords)],
                        pb.at[pl.ds(nwords, nwords)])
        pltpu.sync_copy(w_hbm, wb)
        pltpu.sync_copy(b_hbm, bb)
        bv = bb[...]

        @plsc.parallel_loop(0, nwords // 16, unroll=4)
        def _(u):
            sl = pl.ds(u * 16, 16)
            pb[sl] = _rne_bf16(
                jnp.maximum(pb[sl] + pb[pl.ds(nwords + u * 16, 16)] + bv, 0.0))

        lanes = lax.iota(jnp.int32, 16)
        half = lanes >> 3
        colp = lanes & 7
        h8 = half * D_PAD
        w2r = [wb[pl.ds(16 * k, 16)] for k in range(D_PAD)]

        @plsc.parallel_loop(0, RW // 2, unroll=4)
        def _(r):
            idx = 2 * r * D_PAD + h8
            acc = w2r[0] * plsc.load_gather(pb, [idx])
            for k in range(1, D_PAD):
                idx = idx + 1
                acc = acc + w2r[k] * plsc.load_gather(pb, [idx])
            plsc.store_scatter(ob, [2 * r + half, colp], acc)

        pltpu.sync_copy(ob, out_hbm.at[pl.ds(r0, RW)])

    return mid(p_flat, b1p, w2p)


def _sc_final(p_flat, b2p):
    @functools.partial(
        pl.kernel,
        out_type=jax.ShapeDtypeStruct((N_NODES * D_PAD,), jnp.float32),
        mesh=_MESH,
        scratch_types=[
            pltpu.VMEM((2 * RW * D_PAD,), jnp.float32),
            pltpu.VMEM((16,), jnp.float32),
        ],
        compiler_params=_CP,
    )
    def fin(p_hbm, b_hbm, out_hbm, pb, bb):
        w = lax.axis_index("s") * NC + lax.axis_index("c")
        r0 = w * RW
        nwords = RW * D_PAD
        pltpu.sync_copy(p_hbm.at[pl.ds(r0 * D_PAD, nwords)], pb.at[pl.ds(0, nwords)])
        pltpu.sync_copy(p_hbm.at[pl.ds(N_NODES * D_PAD + r0 * D_PAD, nwords)],
                        pb.at[pl.ds(nwords, nwords)])
        pltpu.sync_copy(b_hbm, bb)
        bv = bb[...]

        @plsc.parallel_loop(0, nwords // 16, unroll=4)
        def _(u):
            sl = pl.ds(u * 16, 16)
            x = pb[sl] + pb[pl.ds(nwords + u * 16, 16)] + bv
            pb[sl] = 1.0 / (1.0 + jnp.exp(-x))

        pltpu.sync_copy(pb.at[pl.ds(0, nwords)],
                        out_hbm.at[pl.ds(r0 * D_PAD, nwords)])

    return fin(p_flat, b2p)


def kernel(adjacency_indices, adjacency_values, conv1, conv2, i, W1, b1, W2, b2):
    kh, kw = conv1.shape[2], conv1.shape[3]
    x1 = conv1.reshape(-1, kh * kw)
    x2 = conv2.reshape(-1, kh * kw)
    feature = jnp.concatenate([x1, x2], axis=0)
    def _noise():
        return jax.random.uniform(jax.random.key(42), feature.shape,
                                  dtype=jnp.float32) * 2.0 - 1.0
    try:
        with jax.ensure_compile_time_eval():
            junyun = _noise()
    except Exception:
        junyun = _noise()
    ij2 = adjacency_indices.reshape(2, N_EDGES // GROUP, GROUP)

    w1_pad = jnp.pad(W1, ((0, 0), (0, D_PAD - HID)))
    w1_pad = w1_pad.astype(jnp.bfloat16).astype(jnp.float32)
    w1p = jnp.concatenate([w1_pad, w1_pad], axis=1).reshape(-1)
    b1p = jnp.tile(jnp.pad(b1, (0, D_PAD - HID)), 2)
    w2_pad = jnp.pad(W2, ((0, D_PAD - HID), (0, D_PAD - OUT_DIM)))
    w2_pad = w2_pad.astype(jnp.bfloat16).astype(jnp.float32)
    w2p = jnp.concatenate([w2_pad, w2_pad], axis=1).reshape(-1)
    b2p = jnp.tile(jnp.pad(b2, (0, D_PAD - OUT_DIM)), 2)
    ivec = jnp.broadcast_to(i, (16,))
    zero_rows = jnp.zeros((ROWS_PER_SUB, D_PAD), jnp.float32)

    support = _sc_layer1(feature.reshape(-1), junyun.reshape(-1), ivec, w1p)
    partials = _spmm_sc(ij2, adjacency_values, support, zero_rows, HID)
    support2 = _sc_mid(partials.reshape(-1), b1p, w2p)
    partials2 = _spmm_sc(ij2, adjacency_values, support2, zero_rows, OUT_DIM)
    out = _sc_final(partials2.reshape(-1), b2p)
    return out.reshape(N_NODES, D_PAD)[:, :OUT_DIM]
